# baseline (speedup 1.0000x reference)
import jax
import jax.numpy as jnp
from jax import lax
from jax.experimental import pallas as pl
from jax.experimental.pallas import tpu as pltpu
from jax.experimental.pallas import tpu_sc as plsc

_N = 10000
_D = 128

_info = plsc.get_sparse_core_info()
_NC, _NS = _info.num_cores, _info.num_subcores
_NW = _NC * _NS
_BP = ((_N + 8 * _NW - 1) // (8 * _NW)) * (8 * _NW)
_B_PER_W = _BP // _NW
_LAST_BASE = (_NW - 1) * _B_PER_W
_LAST_ROWS = _N - _LAST_BASE


def _gather_kernel(table_hbm, idx_hbm, out_hbm, idx_v, rows_v, sem):
    wid = lax.axis_index("s") * _NC + lax.axis_index("c")
    base = wid * _B_PER_W
    pltpu.sync_copy(idx_hbm.at[pl.ds(base, _B_PER_W)], idx_v)
    pltpu.async_copy(table_hbm.at[idx_v], rows_v, sem).wait()

    @pl.when(wid < _NW - 1)
    def _store_full():
        pltpu.sync_copy(rows_v, out_hbm.at[pl.ds(base, _B_PER_W)])

    @pl.when(wid == _NW - 1)
    def _store_tail():
        pltpu.sync_copy(rows_v.at[pl.ds(0, _LAST_ROWS)],
                        out_hbm.at[pl.ds(_LAST_BASE, _LAST_ROWS)])


_mesh = plsc.VectorSubcoreMesh(core_axis_name="c", subcore_axis_name="s")

_gather = pl.kernel(
    _gather_kernel,
    mesh=_mesh,
    out_type=jax.ShapeDtypeStruct((_N, _D), jnp.float32),
    scratch_types=[
        pltpu.VMEM((_B_PER_W,), jnp.int32),
        pltpu.VMEM((_B_PER_W, _D), jnp.float32),
        pltpu.SemaphoreType.DMA,
    ],
)


def kernel(pos, z, neighbours, embedding_table):
    del pos, neighbours
    idx = jnp.concatenate(
        [z.astype(jnp.int32), jnp.zeros((_BP - _N,), dtype=jnp.int32)])
    return _gather(embedding_table, idx)


# speedup vs baseline: 1.5452x; 1.5452x over previous
import jax
import jax.numpy as jnp
from jax import lax
from jax.experimental import pallas as pl
from jax.experimental.pallas import tpu as pltpu
from jax.experimental.pallas import tpu_sc as plsc

_N = 10000
_D = 128

_info = plsc.get_sparse_core_info()
_NC, _NS = _info.num_cores, _info.num_subcores
_NW = _NC * _NS
_B_PER_W = -(-_N // (8 * _NW)) * 8
_CHUNK = 80
_NCHUNK = _B_PER_W // _CHUNK
_LAST_BASE = (_NW - 1) * _B_PER_W
_LAST_ROWS = _N - _LAST_BASE


def _gather_kernel(table_hbm, idx_hbm, out_hbm, idx_v, rows_v, gsems, ssem):
    wid = lax.axis_index("s") * _NC + lax.axis_index("c")
    base = wid * _B_PER_W

    @pl.when(wid < _NW - 1)
    def _full():
        pltpu.sync_copy(idx_hbm.at[pl.ds(base, _B_PER_W)], idx_v)
        copies = []
        for c in range(_NCHUNK):
            cp = pltpu.async_copy(
                table_hbm.at[idx_v.at[pl.ds(c * _CHUNK, _CHUNK)]],
                rows_v.at[c], gsems.at[c])
            copies.append(cp)
        for c in range(_NCHUNK):
            copies[c].wait()
            pltpu.async_copy(
                rows_v.at[c],
                out_hbm.at[pl.ds(base + c * _CHUNK, _CHUNK)], ssem)
        for c in range(_NCHUNK):
            pltpu.make_async_copy(
                rows_v.at[c],
                out_hbm.at[pl.ds(base + c * _CHUNK, _CHUNK)], ssem).wait()

    @pl.when(wid == _NW - 1)
    def _tail():
        pltpu.sync_copy(idx_hbm.at[pl.ds(_LAST_BASE, _LAST_ROWS)],
                        idx_v.at[pl.ds(0, _LAST_ROWS)])
        pltpu.async_copy(
            table_hbm.at[idx_v.at[pl.ds(0, _LAST_ROWS)]],
            rows_v.at[0], gsems.at[0]).wait()
        pltpu.sync_copy(rows_v.at[0],
                        out_hbm.at[pl.ds(_LAST_BASE, _LAST_ROWS)])


_mesh = plsc.VectorSubcoreMesh(core_axis_name="c", subcore_axis_name="s")

_gather = pl.kernel(
    _gather_kernel,
    mesh=_mesh,
    out_type=jax.ShapeDtypeStruct((_N, _D), jnp.float32),
    scratch_types=[
        pltpu.VMEM((_B_PER_W,), jnp.int32),
        pltpu.VMEM((_NCHUNK, _CHUNK, _D), jnp.float32),
        pltpu.SemaphoreType.DMA((_NCHUNK,)),
        pltpu.SemaphoreType.DMA,
    ],
)


def kernel(pos, z, neighbours, embedding_table):
    del pos, neighbours
    return _gather(embedding_table, z.astype(jnp.int32))
